# baseline (speedup 1.0000x reference)
import jax
import jax.numpy as jnp
from jax import lax
from jax.experimental import pallas as pl
from jax.experimental.pallas import tpu as pltpu
from jax.experimental.pallas import tpu_sc as plsc

N_NODES = 10000
N_EDGES = 320000
D_HID = 256

NC = 2
NS = 16
NW = NC * NS
EPW = N_EDGES // NS
NPASS = N_EDGES // EPW
CHUNK = 64
IBLK = 8
NBLK = 22
NCHUNK = NBLK * IBLK
CAPT = NCHUNK * CHUNK
OWN = 320
ACC_ROWS = OWN + 8
N_PAD = NW * OWN
LG = 16

_MESH = plsc.VectorSubcoreMesh(core_axis_name="c", subcore_axis_name="s")
_SC_PARAMS = pltpu.CompilerParams(needs_layout_passes=False)


def _partition_body(src_hbm, dst_hbm, srcl_hbm, dstl_hbm, cnt_hbm,
                    src_v, dst_v, srcc_v, dstc_v, cnt_v):
    c = lax.axis_index("c")
    s = lax.axis_index("s")
    t = s * NC + c
    lo = t * OWN

    zero16 = jnp.zeros((LG,), jnp.int32)
    trash16 = jnp.full((LG,), OWN, jnp.int32)

    def prefill(i, carry):
        srcc_v[pl.ds(i * LG, LG)] = zero16
        dstc_v[pl.ds(i * LG, LG)] = trash16
        return carry

    lax.fori_loop(0, CAPT // LG, prefill, 0)

    def scan_pass(p, off):
        pltpu.sync_copy(src_hbm.at[pl.ds(p * EPW, EPW)], src_v)
        pltpu.sync_copy(dst_hbm.at[pl.ds(p * EPW, EPW)], dst_v)

        def compact(i, off):
            sv0 = src_v[pl.ds(i * 2 * LG, LG)]
            dv0 = dst_v[pl.ds(i * 2 * LG, LG)] - lo
            sv1 = src_v[pl.ds(i * 2 * LG + LG, LG)]
            dv1 = dst_v[pl.ds(i * 2 * LG + LG, LG)] - lo
            keep0 = (dv0 >= 0) & (dv0 < OWN)
            keep1 = (dv1 >= 0) & (dv1 < OWN)
            cs0 = plsc.cumsum(keep0.astype(jnp.int32))
            cs1 = plsc.cumsum(keep1.astype(jnp.int32))
            n0 = cs0[15]
            pos0 = jnp.minimum(off + cs0 - 1, CAPT - 1)
            pos1 = jnp.minimum(off + n0 + cs1 - 1, CAPT - 1)
            plsc.store_scatter(srcc_v, [pos0], sv0, mask=keep0)
            plsc.store_scatter(dstc_v, [pos0], dv0, mask=keep0)
            plsc.store_scatter(srcc_v, [pos1], sv1, mask=keep1)
            plsc.store_scatter(dstc_v, [pos1], dv1, mask=keep1)
            return off + n0 + cs1[15]

        return lax.fori_loop(0, EPW // (2 * LG), compact, off)

    n_kept = lax.fori_loop(0, NPASS, scan_pass, jnp.int32(0))
    n_chunks = jnp.minimum((n_kept + CHUNK - 1) // CHUNK, NCHUNK)

    cnt_v[...] = jnp.full((LG,), n_chunks, jnp.int32)
    pltpu.sync_copy(cnt_v, cnt_hbm.at[t])
    pltpu.sync_copy(srcc_v, srcl_hbm.at[t])
    pltpu.sync_copy(dstc_v, dstl_hbm.at[t])


_partition = pl.kernel(
    _partition_body,
    out_type=(
        jax.ShapeDtypeStruct((NW, CAPT), jnp.int32),
        jax.ShapeDtypeStruct((NW, CAPT), jnp.int32),
        jax.ShapeDtypeStruct((NW, LG), jnp.int32),
    ),
    mesh=_MESH,
    compiler_params=_SC_PARAMS,
    scratch_types=[
        pltpu.VMEM((EPW,), jnp.int32),
        pltpu.VMEM((EPW,), jnp.int32),
        pltpu.VMEM((CAPT,), jnp.int32),
        pltpu.VMEM((CAPT,), jnp.int32),
        pltpu.VMEM((LG,), jnp.int32),
    ],
)


def _agg_body(m_hbm, srcl_hbm, dstl_hbm, cnt_hbm, out_hbm,
              idxs_v, idxd_v, rows_v, cnt_v, acc_v, sem):
    c = lax.axis_index("c")
    s = lax.axis_index("s")
    t = s * NC + c

    zrow = jnp.zeros((LG,), jnp.float32)

    def zfill(i, carry):
        def zcol(k, carry2):
            acc_v[i, pl.ds(k * LG, LG)] = zrow
            return carry2
        return lax.fori_loop(0, D_HID // LG, zcol, carry)

    lax.fori_loop(0, ACC_ROWS, zfill, 0)

    pltpu.sync_copy(cnt_hbm.at[t], cnt_v)
    trips = cnt_v[...][0]

    def fire(j):
        row = lax.rem(j // IBLK, 2)
        off = lax.rem(j, IBLK)
        pltpu.async_copy(
            m_hbm.at[idxs_v.at[row, pl.ds(off * CHUNK, CHUNK)]],
            rows_v.at[lax.rem(j, 2)], sem)

    pltpu.sync_copy(srcl_hbm.at[t, 0], idxs_v.at[0])

    @pl.when(trips > 0)
    def _():
        fire(0)

    def block(b, carry):
        pltpu.sync_copy(srcl_hbm.at[t, jnp.minimum(b + 1, NBLK - 1)],
                        idxs_v.at[lax.rem(b + 1, 2)])
        pltpu.sync_copy(dstl_hbm.at[t, b], idxd_v.at[lax.rem(b, 2)])
        brow = lax.rem(b, 2)

        def step(jj, carry2):
            j = b * IBLK + jj

            @pl.when(j < trips)
            def _():
                pltpu.make_async_copy(
                    m_hbm.at[pl.ds(0, CHUNK)],
                    rows_v.at[lax.rem(j, 2)], sem).wait()

                rrow = lax.rem(j, 2)

                def group(g, carry3):
                    dv = idxd_v[brow, pl.ds(jj * CHUNK + g * LG, LG)]
                    for e in range(LG):
                        d = dv[e]
                        for k in range(D_HID // LG):
                            plsc.addupdate(
                                acc_v.at[d, pl.ds(k * LG, LG)],
                                rows_v[rrow, g * LG + e, pl.ds(k * LG, LG)])
                    return carry3

                lax.fori_loop(0, CHUNK // LG, group, 0)

                @pl.when(j + 1 < trips)
                def _():
                    fire(j + 1)

            return carry2

        return lax.fori_loop(0, IBLK, step, carry)

    lax.fori_loop(0, NBLK, block, 0)

    pltpu.sync_copy(acc_v.at[pl.ds(0, OWN)], out_hbm.at[pl.ds(t * OWN, OWN)])


_aggregate = pl.kernel(
    _agg_body,
    out_type=jax.ShapeDtypeStruct((N_PAD, D_HID), jnp.float32),
    mesh=_MESH,
    compiler_params=_SC_PARAMS,
    scratch_types=[
        pltpu.VMEM((2, IBLK * CHUNK), jnp.int32),
        pltpu.VMEM((2, IBLK * CHUNK), jnp.int32),
        pltpu.VMEM((2, CHUNK, D_HID), jnp.float32),
        pltpu.VMEM((LG,), jnp.int32),
        pltpu.VMEM((ACC_ROWS, D_HID), jnp.float32),
        pltpu.SemaphoreType.DMA,
    ],
)


_BLK = 1000


def _mm_body(x_ref, w_ref, o_ref):
    o_ref[...] = jnp.dot(x_ref[...], w_ref[...],
                         precision=lax.Precision.HIGHEST,
                         preferred_element_type=jnp.float32)


def _mm_bias_relu_body(a_ref, b_ref, w_ref, o_ref):
    h = jnp.maximum(a_ref[...] + b_ref[...], 0.0)
    o_ref[...] = jnp.dot(h, w_ref[...],
                         precision=lax.Precision.HIGHEST,
                         preferred_element_type=jnp.float32)


def _bias_relu_body(a_ref, b_ref, o_ref):
    o_ref[...] = jnp.maximum(a_ref[...] + b_ref[...], 0.0)


def _matmul(x, w):
    d_in = x.shape[1]
    return pl.pallas_call(
        _mm_body,
        grid=(N_NODES // _BLK,),
        in_specs=[
            pl.BlockSpec((_BLK, d_in), lambda i: (i, 0)),
            pl.BlockSpec((d_in, D_HID), lambda i: (0, 0)),
        ],
        out_specs=pl.BlockSpec((_BLK, D_HID), lambda i: (i, 0)),
        out_shape=jax.ShapeDtypeStruct((N_NODES, D_HID), jnp.float32),
    )(x, w)


def _matmul_bias_relu(a, b, w):
    return pl.pallas_call(
        _mm_bias_relu_body,
        grid=(N_NODES // _BLK,),
        in_specs=[
            pl.BlockSpec((_BLK, D_HID), lambda i: (i, 0)),
            pl.BlockSpec((1, D_HID), lambda i: (0, 0)),
            pl.BlockSpec((D_HID, D_HID), lambda i: (0, 0)),
        ],
        out_specs=pl.BlockSpec((_BLK, D_HID), lambda i: (i, 0)),
        out_shape=jax.ShapeDtypeStruct((N_NODES, D_HID), jnp.float32),
    )(a, b.reshape(1, D_HID), w)


def _bias_relu(a, b):
    return pl.pallas_call(
        _bias_relu_body,
        grid=(N_NODES // _BLK,),
        in_specs=[
            pl.BlockSpec((_BLK, D_HID), lambda i: (i, 0)),
            pl.BlockSpec((1, D_HID), lambda i: (0, 0)),
        ],
        out_specs=pl.BlockSpec((_BLK, D_HID), lambda i: (i, 0)),
        out_shape=jax.ShapeDtypeStruct((N_NODES, D_HID), jnp.float32),
    )(a, b.reshape(1, D_HID))


def kernel(x, edge_index, W1, b1, W2, b2, W3, b3):
    src = edge_index[0].astype(jnp.int32)
    dst = edge_index[1].astype(jnp.int32)

    srcl, dstl, cnt = _partition(src, dst)
    srcl = srcl.reshape(NW, NBLK, IBLK * CHUNK)
    dstl = dstl.reshape(NW, NBLK, IBLK * CHUNK)

    def gcn_agg(m):
        agg = _aggregate(m, srcl, dstl, cnt)
        return agg[:N_NODES]

    agg1 = gcn_agg(_matmul(x, W1))
    agg2 = gcn_agg(_matmul_bias_relu(agg1, b1, W2))
    agg3 = gcn_agg(_matmul_bias_relu(agg2, b2, W3))
    return _bias_relu(agg3, b3)


# speedup vs baseline: 1.2540x; 1.2540x over previous
import jax
import jax.numpy as jnp
from jax import lax
from jax.experimental import pallas as pl
from jax.experimental.pallas import tpu as pltpu
from jax.experimental.pallas import tpu_sc as plsc

N_NODES = 10000
N_EDGES = 320000
D_HID = 256

NC = 2
NS = 16
NW = NC * NS
EPW = N_EDGES // NS
NPASS = N_EDGES // EPW
CHUNK = 64
IBLK = 8
NBLK = 22
NCHUNK = NBLK * IBLK
CAPT = NCHUNK * CHUNK
OWN = 320
ACC_ROWS = OWN + 8
N_PAD = NW * OWN
LG = 16

_MESH = plsc.VectorSubcoreMesh(core_axis_name="c", subcore_axis_name="s")
_SC_PARAMS = pltpu.CompilerParams(needs_layout_passes=False)


def _partition_body(src_hbm, dst_hbm, srcl_hbm, dstl_hbm, cnt_hbm,
                    src_v, dst_v, srcc_v, dstc_v, cnt_v):
    c = lax.axis_index("c")
    s = lax.axis_index("s")
    t = s * NC + c
    lo = t * OWN

    zero16 = jnp.zeros((LG,), jnp.int32)
    trash16 = jnp.full((LG,), OWN, jnp.int32)

    def prefill(i, carry):
        srcc_v[pl.ds(i * LG, LG)] = zero16
        dstc_v[pl.ds(i * LG, LG)] = trash16
        return carry

    lax.fori_loop(0, CAPT // LG, prefill, 0)

    def scan_pass(p, off):
        pltpu.sync_copy(src_hbm.at[pl.ds(p * EPW, EPW)], src_v)
        pltpu.sync_copy(dst_hbm.at[pl.ds(p * EPW, EPW)], dst_v)

        def compact(i, off):
            sv0 = src_v[pl.ds(i * 2 * LG, LG)]
            dv0 = dst_v[pl.ds(i * 2 * LG, LG)] - lo
            sv1 = src_v[pl.ds(i * 2 * LG + LG, LG)]
            dv1 = dst_v[pl.ds(i * 2 * LG + LG, LG)] - lo
            keep0 = (dv0 >= 0) & (dv0 < OWN)
            keep1 = (dv1 >= 0) & (dv1 < OWN)
            cs0 = plsc.cumsum(keep0.astype(jnp.int32))
            cs1 = plsc.cumsum(keep1.astype(jnp.int32))
            n0 = cs0[15]
            pos0 = jnp.minimum(off + cs0 - 1, CAPT - 1)
            pos1 = jnp.minimum(off + n0 + cs1 - 1, CAPT - 1)
            plsc.store_scatter(srcc_v, [pos0], sv0, mask=keep0)
            plsc.store_scatter(dstc_v, [pos0], dv0, mask=keep0)
            plsc.store_scatter(srcc_v, [pos1], sv1, mask=keep1)
            plsc.store_scatter(dstc_v, [pos1], dv1, mask=keep1)
            return off + n0 + cs1[15]

        return lax.fori_loop(0, EPW // (2 * LG), compact, off)

    n_kept = lax.fori_loop(0, NPASS, scan_pass, jnp.int32(0))
    n_chunks = jnp.minimum((n_kept + CHUNK - 1) // CHUNK, NCHUNK)

    cnt_v[...] = jnp.full((LG,), n_chunks, jnp.int32)
    pltpu.sync_copy(cnt_v, cnt_hbm.at[t])
    pltpu.sync_copy(srcc_v, srcl_hbm.at[t])
    pltpu.sync_copy(dstc_v, dstl_hbm.at[t])


_partition = pl.kernel(
    _partition_body,
    out_type=(
        jax.ShapeDtypeStruct((NW, CAPT), jnp.int32),
        jax.ShapeDtypeStruct((NW, CAPT), jnp.int32),
        jax.ShapeDtypeStruct((NW, LG), jnp.int32),
    ),
    mesh=_MESH,
    compiler_params=_SC_PARAMS,
    scratch_types=[
        pltpu.VMEM((EPW,), jnp.int32),
        pltpu.VMEM((EPW,), jnp.int32),
        pltpu.VMEM((CAPT,), jnp.int32),
        pltpu.VMEM((CAPT,), jnp.int32),
        pltpu.VMEM((LG,), jnp.int32),
    ],
)


def _agg_body(m_hbm, srcl_hbm, dstl_hbm, cnt_hbm, out_hbm,
              idxs_v, idxd_v, rows_v, cnt_v, acc_v, sem):
    c = lax.axis_index("c")
    s = lax.axis_index("s")
    t = s * NC + c

    zrow = jnp.zeros((LG,), jnp.float32)

    def zfill(i, carry):
        def zcol(k, carry2):
            acc_v[i, pl.ds(k * LG, LG)] = zrow
            return carry2
        return lax.fori_loop(0, D_HID // LG, zcol, carry)

    lax.fori_loop(0, ACC_ROWS, zfill, 0)

    pltpu.sync_copy(cnt_hbm.at[t], cnt_v)
    trips = cnt_v[...][0]

    def fire(j):
        row = lax.rem(j // IBLK, 2)
        off = lax.rem(j, IBLK)
        pltpu.async_copy(
            m_hbm.at[idxs_v.at[row, pl.ds(off * CHUNK, CHUNK)]],
            rows_v.at[lax.rem(j, 2)], sem)

    pltpu.sync_copy(srcl_hbm.at[t, 0], idxs_v.at[0])

    @pl.when(trips > 0)
    def _():
        fire(0)

    def block(b, carry):
        pltpu.sync_copy(srcl_hbm.at[t, jnp.minimum(b + 1, NBLK - 1)],
                        idxs_v.at[lax.rem(b + 1, 2)])
        pltpu.sync_copy(dstl_hbm.at[t, b], idxd_v.at[lax.rem(b, 2)])
        brow = lax.rem(b, 2)

        def step(jj, carry2):
            j = b * IBLK + jj

            @pl.when(j < trips)
            def _():
                pltpu.make_async_copy(
                    m_hbm.at[pl.ds(0, CHUNK)],
                    rows_v.at[lax.rem(j, 2)], sem).wait()

                @pl.when(j + 1 < trips)
                def _():
                    fire(j + 1)

                rrow = lax.rem(j, 2)

                @plsc.parallel_loop(0, CHUNK // LG)
                def group(g):
                    dv = idxd_v[brow, pl.ds(jj * CHUNK + g * LG, LG)]
                    for e in range(LG):
                        d = dv[e]
                        for k in range(D_HID // LG):
                            plsc.addupdate(
                                acc_v.at[d, pl.ds(k * LG, LG)],
                                rows_v[rrow, g * LG + e, pl.ds(k * LG, LG)])

            return carry2

        return lax.fori_loop(0, IBLK, step, carry)

    lax.fori_loop(0, NBLK, block, 0)

    pltpu.sync_copy(acc_v.at[pl.ds(0, OWN)], out_hbm.at[pl.ds(t * OWN, OWN)])


_aggregate = pl.kernel(
    _agg_body,
    out_type=jax.ShapeDtypeStruct((N_PAD, D_HID), jnp.float32),
    mesh=_MESH,
    compiler_params=_SC_PARAMS,
    scratch_types=[
        pltpu.VMEM((2, IBLK * CHUNK), jnp.int32),
        pltpu.VMEM((2, IBLK * CHUNK), jnp.int32),
        pltpu.VMEM((2, CHUNK, D_HID), jnp.float32),
        pltpu.VMEM((LG,), jnp.int32),
        pltpu.VMEM((ACC_ROWS, D_HID), jnp.float32),
        pltpu.SemaphoreType.DMA,
    ],
)


_BLK = 1000


def _mm_body(x_ref, w_ref, o_ref):
    o_ref[...] = jnp.dot(x_ref[...], w_ref[...],
                         precision=lax.Precision.HIGHEST,
                         preferred_element_type=jnp.float32)


def _mm_bias_relu_body(a_ref, b_ref, w_ref, o_ref):
    h = jnp.maximum(a_ref[...] + b_ref[...], 0.0)
    o_ref[...] = jnp.dot(h, w_ref[...],
                         precision=lax.Precision.HIGHEST,
                         preferred_element_type=jnp.float32)


def _bias_relu_body(a_ref, b_ref, o_ref):
    o_ref[...] = jnp.maximum(a_ref[...] + b_ref[...], 0.0)


def _matmul(x, w):
    d_in = x.shape[1]
    return pl.pallas_call(
        _mm_body,
        grid=(N_NODES // _BLK,),
        in_specs=[
            pl.BlockSpec((_BLK, d_in), lambda i: (i, 0)),
            pl.BlockSpec((d_in, D_HID), lambda i: (0, 0)),
        ],
        out_specs=pl.BlockSpec((_BLK, D_HID), lambda i: (i, 0)),
        out_shape=jax.ShapeDtypeStruct((N_NODES, D_HID), jnp.float32),
    )(x, w)


def _matmul_bias_relu(a, b, w):
    return pl.pallas_call(
        _mm_bias_relu_body,
        grid=(N_NODES // _BLK,),
        in_specs=[
            pl.BlockSpec((_BLK, D_HID), lambda i: (i, 0)),
            pl.BlockSpec((1, D_HID), lambda i: (0, 0)),
            pl.BlockSpec((D_HID, D_HID), lambda i: (0, 0)),
        ],
        out_specs=pl.BlockSpec((_BLK, D_HID), lambda i: (i, 0)),
        out_shape=jax.ShapeDtypeStruct((N_NODES, D_HID), jnp.float32),
    )(a, b.reshape(1, D_HID), w)


def _bias_relu(a, b):
    return pl.pallas_call(
        _bias_relu_body,
        grid=(N_NODES // _BLK,),
        in_specs=[
            pl.BlockSpec((_BLK, D_HID), lambda i: (i, 0)),
            pl.BlockSpec((1, D_HID), lambda i: (0, 0)),
        ],
        out_specs=pl.BlockSpec((_BLK, D_HID), lambda i: (i, 0)),
        out_shape=jax.ShapeDtypeStruct((N_NODES, D_HID), jnp.float32),
    )(a, b.reshape(1, D_HID))


def kernel(x, edge_index, W1, b1, W2, b2, W3, b3):
    src = edge_index[0].astype(jnp.int32)
    dst = edge_index[1].astype(jnp.int32)

    srcl, dstl, cnt = _partition(src, dst)
    srcl = srcl.reshape(NW, NBLK, IBLK * CHUNK)
    dstl = dstl.reshape(NW, NBLK, IBLK * CHUNK)

    def gcn_agg(m):
        agg = _aggregate(m, srcl, dstl, cnt)
        return agg[:N_NODES]

    agg1 = gcn_agg(_matmul(x, W1))
    agg2 = gcn_agg(_matmul_bias_relu(agg1, b1, W2))
    agg3 = gcn_agg(_matmul_bias_relu(agg2, b2, W3))
    return _bias_relu(agg3, b3)


# speedup vs baseline: 1.2566x; 1.0020x over previous
import jax
import jax.numpy as jnp
from jax import lax
from jax.experimental import pallas as pl
from jax.experimental.pallas import tpu as pltpu
from jax.experimental.pallas import tpu_sc as plsc

N_NODES = 10000
N_EDGES = 320000
D_HID = 256

NC = 2
NS = 16
NW = NC * NS
EPW = N_EDGES // NS
NPASS = N_EDGES // EPW
CHUNK = 64
IBLK = 8
NBLK = 22
NCHUNK = NBLK * IBLK
CAPT = NCHUNK * CHUNK
OWN = 320
ACC_ROWS = OWN + 8
N_PAD = NW * OWN
LG = 16

_MESH = plsc.VectorSubcoreMesh(core_axis_name="c", subcore_axis_name="s")
_SC_PARAMS = pltpu.CompilerParams(needs_layout_passes=False)


def _partition_body(src_hbm, dst_hbm, srcl_hbm, dstl_hbm, cnt_hbm,
                    src_v, dst_v, srcc_v, dstc_v, cnt_v):
    c = lax.axis_index("c")
    s = lax.axis_index("s")
    t = s * NC + c
    lo = t * OWN

    zero16 = jnp.zeros((LG,), jnp.int32)
    trash16 = jnp.full((LG,), OWN, jnp.int32)

    def prefill(i, carry):
        srcc_v[pl.ds(i * LG, LG)] = zero16
        dstc_v[pl.ds(i * LG, LG)] = trash16
        return carry

    lax.fori_loop(0, CAPT // LG, prefill, 0)

    def scan_pass(p, off):
        pltpu.sync_copy(src_hbm.at[pl.ds(p * EPW, EPW)], src_v)
        pltpu.sync_copy(dst_hbm.at[pl.ds(p * EPW, EPW)], dst_v)

        def compact(i, off):
            sv0 = src_v[pl.ds(i * 2 * LG, LG)]
            dv0 = dst_v[pl.ds(i * 2 * LG, LG)] - lo
            sv1 = src_v[pl.ds(i * 2 * LG + LG, LG)]
            dv1 = dst_v[pl.ds(i * 2 * LG + LG, LG)] - lo
            keep0 = (dv0 >= 0) & (dv0 < OWN)
            keep1 = (dv1 >= 0) & (dv1 < OWN)
            cs0 = plsc.cumsum(keep0.astype(jnp.int32))
            cs1 = plsc.cumsum(keep1.astype(jnp.int32))
            n0 = cs0[15]
            pos0 = jnp.minimum(off + cs0 - 1, CAPT - 1)
            pos1 = jnp.minimum(off + n0 + cs1 - 1, CAPT - 1)
            plsc.store_scatter(srcc_v, [pos0], sv0, mask=keep0)
            plsc.store_scatter(dstc_v, [pos0], dv0, mask=keep0)
            plsc.store_scatter(srcc_v, [pos1], sv1, mask=keep1)
            plsc.store_scatter(dstc_v, [pos1], dv1, mask=keep1)
            return off + n0 + cs1[15]

        return lax.fori_loop(0, EPW // (2 * LG), compact, off)

    n_kept = lax.fori_loop(0, NPASS, scan_pass, jnp.int32(0))
    n_chunks = jnp.minimum((n_kept + CHUNK - 1) // CHUNK, NCHUNK)

    cnt_v[...] = jnp.full((LG,), n_chunks, jnp.int32)
    pltpu.sync_copy(cnt_v, cnt_hbm.at[t])
    pltpu.sync_copy(srcc_v, srcl_hbm.at[t])
    pltpu.sync_copy(dstc_v, dstl_hbm.at[t])


_partition = pl.kernel(
    _partition_body,
    out_type=(
        jax.ShapeDtypeStruct((NW, CAPT), jnp.int32),
        jax.ShapeDtypeStruct((NW, CAPT), jnp.int32),
        jax.ShapeDtypeStruct((NW, LG), jnp.int32),
    ),
    mesh=_MESH,
    compiler_params=_SC_PARAMS,
    scratch_types=[
        pltpu.VMEM((EPW,), jnp.int32),
        pltpu.VMEM((EPW,), jnp.int32),
        pltpu.VMEM((CAPT,), jnp.int32),
        pltpu.VMEM((CAPT,), jnp.int32),
        pltpu.VMEM((LG,), jnp.int32),
    ],
)


def _agg_body(m_hbm, srcl_hbm, dstl_hbm, cnt_hbm, out_hbm,
              idxs_v, idxd_v, rows_v, cnt_v, acc_v, sem):
    c = lax.axis_index("c")
    s = lax.axis_index("s")
    t = s * NC + c

    zrow = jnp.zeros((LG,), jnp.float32)

    def zfill(i, carry):
        def zcol(k, carry2):
            acc_v[i, pl.ds(k * LG, LG)] = zrow
            return carry2
        return lax.fori_loop(0, D_HID // LG, zcol, carry)

    lax.fori_loop(0, ACC_ROWS, zfill, 0)

    pltpu.sync_copy(cnt_hbm.at[t], cnt_v)
    trips = cnt_v[...][0]

    def fire(j):
        row = lax.rem(j // IBLK, 2)
        off = lax.rem(j, IBLK)
        pltpu.async_copy(
            m_hbm.at[idxs_v.at[row, pl.ds(off * CHUNK, CHUNK)]],
            rows_v.at[lax.rem(j, 2)], sem)

    pltpu.sync_copy(srcl_hbm.at[t, 0], idxs_v.at[0])

    @pl.when(trips > 0)
    def _():
        fire(0)

    def block(b, carry):
        pltpu.sync_copy(srcl_hbm.at[t, jnp.minimum(b + 1, NBLK - 1)],
                        idxs_v.at[lax.rem(b + 1, 2)])
        pltpu.sync_copy(dstl_hbm.at[t, b], idxd_v.at[lax.rem(b, 2)])
        brow = lax.rem(b, 2)

        def step(jj, carry2):
            j = b * IBLK + jj

            @pl.when(j < trips)
            def _():
                pltpu.make_async_copy(
                    m_hbm.at[pl.ds(0, CHUNK)],
                    rows_v.at[lax.rem(j, 2)], sem).wait()

                @pl.when(j + 1 < trips)
                def _():
                    fire(j + 1)

                rrow = lax.rem(j, 2)

                @plsc.parallel_loop(0, CHUNK // LG, unroll=2)
                def group(g):
                    dv = idxd_v[brow, pl.ds(jj * CHUNK + g * LG, LG)]
                    for e in range(LG):
                        d = dv[e]
                        for k in range(D_HID // LG):
                            plsc.addupdate(
                                acc_v.at[d, pl.ds(k * LG, LG)],
                                rows_v[rrow, g * LG + e, pl.ds(k * LG, LG)])

            return carry2

        return lax.fori_loop(0, IBLK, step, carry)

    lax.fori_loop(0, NBLK, block, 0)

    pltpu.sync_copy(acc_v.at[pl.ds(0, OWN)], out_hbm.at[pl.ds(t * OWN, OWN)])


_aggregate = pl.kernel(
    _agg_body,
    out_type=jax.ShapeDtypeStruct((N_PAD, D_HID), jnp.float32),
    mesh=_MESH,
    compiler_params=_SC_PARAMS,
    scratch_types=[
        pltpu.VMEM((2, IBLK * CHUNK), jnp.int32),
        pltpu.VMEM((2, IBLK * CHUNK), jnp.int32),
        pltpu.VMEM((2, CHUNK, D_HID), jnp.float32),
        pltpu.VMEM((LG,), jnp.int32),
        pltpu.VMEM((ACC_ROWS, D_HID), jnp.float32),
        pltpu.SemaphoreType.DMA,
    ],
)


_BLK = 1000


def _mm_body(x_ref, w_ref, o_ref):
    o_ref[...] = jnp.dot(x_ref[...], w_ref[...],
                         precision=lax.Precision.HIGHEST,
                         preferred_element_type=jnp.float32)


def _mm_bias_relu_body(a_ref, b_ref, w_ref, o_ref):
    h = jnp.maximum(a_ref[...] + b_ref[...], 0.0)
    o_ref[...] = jnp.dot(h, w_ref[...],
                         precision=lax.Precision.HIGHEST,
                         preferred_element_type=jnp.float32)


def _bias_relu_body(a_ref, b_ref, o_ref):
    o_ref[...] = jnp.maximum(a_ref[...] + b_ref[...], 0.0)


def _matmul(x, w):
    d_in = x.shape[1]
    return pl.pallas_call(
        _mm_body,
        grid=(N_NODES // _BLK,),
        in_specs=[
            pl.BlockSpec((_BLK, d_in), lambda i: (i, 0)),
            pl.BlockSpec((d_in, D_HID), lambda i: (0, 0)),
        ],
        out_specs=pl.BlockSpec((_BLK, D_HID), lambda i: (i, 0)),
        out_shape=jax.ShapeDtypeStruct((N_NODES, D_HID), jnp.float32),
    )(x, w)


def _matmul_bias_relu(a, b, w):
    return pl.pallas_call(
        _mm_bias_relu_body,
        grid=(N_NODES // _BLK,),
        in_specs=[
            pl.BlockSpec((_BLK, D_HID), lambda i: (i, 0)),
            pl.BlockSpec((1, D_HID), lambda i: (0, 0)),
            pl.BlockSpec((D_HID, D_HID), lambda i: (0, 0)),
        ],
        out_specs=pl.BlockSpec((_BLK, D_HID), lambda i: (i, 0)),
        out_shape=jax.ShapeDtypeStruct((N_NODES, D_HID), jnp.float32),
    )(a, b.reshape(1, D_HID), w)


def _bias_relu(a, b):
    return pl.pallas_call(
        _bias_relu_body,
        grid=(N_NODES // _BLK,),
        in_specs=[
            pl.BlockSpec((_BLK, D_HID), lambda i: (i, 0)),
            pl.BlockSpec((1, D_HID), lambda i: (0, 0)),
        ],
        out_specs=pl.BlockSpec((_BLK, D_HID), lambda i: (i, 0)),
        out_shape=jax.ShapeDtypeStruct((N_NODES, D_HID), jnp.float32),
    )(a, b.reshape(1, D_HID))


def kernel(x, edge_index, W1, b1, W2, b2, W3, b3):
    src = edge_index[0].astype(jnp.int32)
    dst = edge_index[1].astype(jnp.int32)

    srcl, dstl, cnt = _partition(src, dst)
    srcl = srcl.reshape(NW, NBLK, IBLK * CHUNK)
    dstl = dstl.reshape(NW, NBLK, IBLK * CHUNK)

    def gcn_agg(m):
        agg = _aggregate(m, srcl, dstl, cnt)
        return agg[:N_NODES]

    agg1 = gcn_agg(_matmul(x, W1))
    agg2 = gcn_agg(_matmul_bias_relu(agg1, b1, W2))
    agg3 = gcn_agg(_matmul_bias_relu(agg2, b2, W3))
    return _bias_relu(agg3, b3)
